# Initial kernel scaffold; baseline (speedup 1.0000x reference)
import jax
import jax.numpy as jnp
from jax.experimental import pallas as pl


def kernel(x, W_qkv, b_qkv, W_proj, b_proj):
    raise NotImplementedError("write your pallas kernel here")


# speedup vs baseline: 1.1600x; 1.1600x over previous
import functools

import jax
import jax.numpy as jnp
from jax.experimental import pallas as pl

_HEADS = 16


def _matmul_bias_kernel(x_ref, w_ref, b_ref, o_ref):
    acc = jax.lax.dot_general(
        x_ref[...], w_ref[...],
        dimension_numbers=(((1,), (1,)), ((), ())),
        preferred_element_type=jnp.float32,
    )
    o_ref[...] = acc + b_ref[...]


def _matmul_bias(x, w, b, bm, bn):
    M, K = x.shape
    N = w.shape[0]
    return pl.pallas_call(
        _matmul_bias_kernel,
        grid=(M // bm, N // bn),
        in_specs=[
            pl.BlockSpec((bm, K), lambda i, j: (i, 0)),
            pl.BlockSpec((bn, K), lambda i, j: (j, 0)),
            pl.BlockSpec((1, bn), lambda i, j: (0, j)),
        ],
        out_specs=pl.BlockSpec((bm, bn), lambda i, j: (i, j)),
        out_shape=jax.ShapeDtypeStruct((M, N), jnp.float32),
    )(x, w, b.reshape(1, N))


def _attn_kernel(q_ref, k_ref, v_ref, o_ref, *, scale):
    q = q_ref[0]
    k = k_ref[0]
    v = v_ref[0]
    s = jax.lax.dot_general(
        q * scale, k,
        dimension_numbers=(((1,), (1,)), ((), ())),
        preferred_element_type=jnp.float32,
    )
    m = jnp.max(s, axis=-1, keepdims=True)
    p = jnp.exp(s - m)
    l = jnp.sum(p, axis=-1, keepdims=True)
    o = jnp.dot(p, v, preferred_element_type=jnp.float32)
    o_ref[0] = o / l


def _attention(q, k, v, scale, bq):
    BH, N, Dh = q.shape
    return pl.pallas_call(
        functools.partial(_attn_kernel, scale=scale),
        grid=(BH, N // bq),
        in_specs=[
            pl.BlockSpec((1, bq, Dh), lambda bh, qi: (bh, qi, 0)),
            pl.BlockSpec((1, N, Dh), lambda bh, qi: (bh, 0, 0)),
            pl.BlockSpec((1, N, Dh), lambda bh, qi: (bh, 0, 0)),
        ],
        out_specs=pl.BlockSpec((1, bq, Dh), lambda bh, qi: (bh, qi, 0)),
        out_shape=jax.ShapeDtypeStruct((BH, N, Dh), jnp.float32),
    )(q, k, v)


def kernel(x, W_qkv, b_qkv, W_proj, b_proj):
    B, N, C = x.shape
    H = _HEADS
    Dh = C // H
    scale = Dh ** (-0.5)

    qkv = _matmul_bias(x.reshape(B * N, C), W_qkv, b_qkv, bm=512, bn=1024)
    qkv = qkv.reshape(B, N, 3, H, Dh).transpose(2, 0, 3, 1, 4)
    qkv = qkv.reshape(3, B * H, N, Dh)
    q, k, v = qkv[0], qkv[1], qkv[2]

    o = _attention(q, k, v, scale, bq=512)
    o = o.reshape(B, H, N, Dh).transpose(0, 2, 1, 3).reshape(B * N, C)

    out = _matmul_bias(o, W_proj, b_proj, bm=512, bn=1024)
    return out.reshape(B, N, C)
